# Initial kernel scaffold; baseline (speedup 1.0000x reference)
import jax
import jax.numpy as jnp
from jax.experimental import pallas as pl


def kernel(nodes, positions, query_positions, params):
    raise NotImplementedError("write your pallas kernel here")


# speedup vs baseline: 2.9560x; 2.9560x over previous
import jax
import jax.numpy as jnp
from jax.experimental import pallas as pl

KNN = 3
NQ = 10000
NS = 10000
NQP = 10240
NSP = 10240
BQ = 256


def _knn_body(qb_ref, pt_ref, q2_ref, p2_ref, idx_ref, nd_ref):
    qb = qb_ref[...]
    pt = pt_ref[...]
    qp = jax.lax.dot_general(qb, pt, (((1,), (0,)), ((), ())),
                             preferred_element_type=jnp.float32)
    d2 = (q2_ref[...] + p2_ref[...]) - 2.0 * qp
    x = jnp.sqrt(jnp.maximum(d2, 1e-12))
    col = jax.lax.broadcasted_iota(jnp.int32, (BQ, NSP), 1)
    idxs, nds = [], []
    for _ in range(KNN):
        m = jnp.min(x, axis=1, keepdims=True)
        j = jnp.min(jnp.where(x == m, col, NSP), axis=1, keepdims=True)
        idxs.append(j)
        nds.append(m)
        x = jnp.where(col == j, jnp.float32(jnp.inf), x)
    idx_ref[...] = jnp.concatenate(idxs, axis=1)
    nd_ref[...] = jnp.concatenate(nds, axis=1)


def _knn(query_positions, positions):
    qb = jnp.pad(query_positions.astype(jnp.bfloat16),
                 ((0, NQP - NQ), (0, 5)))
    pt = jnp.pad(positions.astype(jnp.bfloat16),
                 ((0, NSP - NS), (0, 5))).T
    q2 = jnp.pad(jnp.sum(query_positions ** 2, axis=1),
                 (0, NQP - NQ)).reshape(NQP, 1)
    p2 = jnp.pad(jnp.sum(positions ** 2, axis=1),
                 (0, NSP - NS), constant_values=1e12).reshape(1, NSP)
    idx, nd = pl.pallas_call(
        _knn_body,
        grid=(NQP // BQ,),
        in_specs=[
            pl.BlockSpec((BQ, 8), lambda i: (i, 0)),
            pl.BlockSpec((8, NSP), lambda i: (0, 0)),
            pl.BlockSpec((BQ, 1), lambda i: (i, 0)),
            pl.BlockSpec((1, NSP), lambda i: (0, 0)),
        ],
        out_specs=[
            pl.BlockSpec((BQ, KNN), lambda i: (i, 0)),
            pl.BlockSpec((BQ, KNN), lambda i: (i, 0)),
        ],
        out_shape=[
            jax.ShapeDtypeStruct((NQP, KNN), jnp.int32),
            jax.ShapeDtypeStruct((NQP, KNN), jnp.float32),
        ],
    )(qb, pt, q2, p2)
    return idx[:NQ], nd[:NQ]


def _mlp_apply(x, Ws, bs):
    n = len(Ws)
    for i in range(n):
        x = x @ Ws[i] + bs[i]
        if i < n - 1:
            x = jax.nn.gelu(x, approximate=False)
    return x


def kernel(nodes, positions, query_positions, params):
    nq = query_positions.shape[0]
    nearest_idx, nearest_d = _knn(query_positions, positions)
    receivers = jnp.repeat(jnp.arange(nq), KNN)
    senders = nearest_idx.reshape(-1)
    edge_feats = jnp.broadcast_to(nearest_d.reshape(-1, 1), (nq * KNN, KNN))
    edges = _mlp_apply(edge_feats, params["enc_W"], params["enc_b"])
    probe_nodes = jnp.zeros((nq, nodes.shape[1]), nodes.dtype).at[receivers].add(nodes[senders])
    counts = jnp.zeros((nq,), jnp.float32).at[receivers].add(jnp.ones((nq * KNN,), jnp.float32))
    probe_nodes = probe_nodes / (counts[:, None] + 1e-8)
    pn = probe_nodes
    for lp in params["layers"]:
        sf = pn[senders]
        rf = pn[receivers]
        ein = jnp.concatenate([sf, rf, edges], axis=-1)
        new_edges = _mlp_apply(ein, lp["eW"], lp["eb"])
        agg = jnp.zeros((pn.shape[0], new_edges.shape[1]), pn.dtype).at[receivers].add(new_edges)
        nin = jnp.concatenate([pn, agg], axis=-1)
        new_nodes = _mlp_apply(nin, lp["nW"], lp["nb"])
        pn = pn + new_nodes
        edges = edges + new_edges
    return _mlp_apply(pn, params["out_W"], params["out_b"])


# speedup vs baseline: 4.4966x; 1.5212x over previous
import jax
import jax.numpy as jnp
from jax.experimental import pallas as pl

KNN = 3
NQ = 10000
NS = 10000
NQP = 10240
NSP = 10240
BQ = 256
BD = 512
ND = 128
ED = 16


def _knn_body(qb_ref, pt_ref, q2_ref, p2_ref, idx_ref, nd_ref):
    qb = qb_ref[...]
    pt = pt_ref[...]
    qp = jax.lax.dot_general(qb, pt, (((1,), (0,)), ((), ())),
                             preferred_element_type=jnp.float32)
    d2 = (q2_ref[...] + p2_ref[...]) - 2.0 * qp
    x = jnp.sqrt(jnp.maximum(d2, 1e-12))
    col = jax.lax.broadcasted_iota(jnp.int32, (BQ, NSP), 1)
    idxs, nds = [], []
    for _ in range(KNN):
        m = jnp.min(x, axis=1, keepdims=True)
        j = jnp.min(jnp.where(x == m, col, NSP), axis=1, keepdims=True)
        idxs.append(j)
        nds.append(m)
        x = jnp.where(col == j, jnp.float32(jnp.inf), x)
    idx_ref[...] = jnp.concatenate(idxs, axis=1)
    nd_ref[...] = jnp.concatenate(nds, axis=1)


def _knn(query_positions, positions):
    qb = jnp.pad(query_positions.astype(jnp.bfloat16),
                 ((0, NQP - NQ), (0, 5)))
    pt = jnp.pad(positions.astype(jnp.bfloat16),
                 ((0, NSP - NS), (0, 5))).T
    q2 = jnp.pad(jnp.sum(query_positions ** 2, axis=1),
                 (0, NQP - NQ)).reshape(NQP, 1)
    p2 = jnp.pad(jnp.sum(positions ** 2, axis=1),
                 (0, NSP - NS), constant_values=1e12).reshape(1, NSP)
    return pl.pallas_call(
        _knn_body,
        grid=(NQP // BQ,),
        in_specs=[
            pl.BlockSpec((BQ, 8), lambda i: (i, 0)),
            pl.BlockSpec((8, NSP), lambda i: (0, 0)),
            pl.BlockSpec((BQ, 1), lambda i: (i, 0)),
            pl.BlockSpec((1, NSP), lambda i: (0, 0)),
        ],
        out_specs=[
            pl.BlockSpec((BQ, KNN), lambda i: (i, 0)),
            pl.BlockSpec((BQ, KNN), lambda i: (i, 0)),
        ],
        out_shape=[
            jax.ShapeDtypeStruct((NQP, KNN), jnp.int32),
            jax.ShapeDtypeStruct((NQP, KNN), jnp.float32),
        ],
    )(qb, pt, q2, p2)


_SQRT_HALF = 0.7071067811865476


def _gelu(x):
    return 0.5 * x * (1.0 + jax.lax.erf(x * _SQRT_HALF))


def _bdot(a, b):
    return jax.lax.dot_general(a.astype(jnp.bfloat16), b,
                               (((1,), (0,)), ((), ())),
                               preferred_element_type=jnp.float32)


def _pre_body(g_ref, ndb_ref, w1s_ref, b1_ref, w2_ref, b2_ref,
              pn_ref, e0_ref, e1_ref, e2_ref):
    g = g_ref[...]
    s = (g[:, :ND] + g[:, ND:2 * ND]) + g[:, 2 * ND:3 * ND]
    pn_ref[...] = s / 3.0
    ndb = ndb_ref[...].astype(jnp.float32)
    w1s = w1s_ref[...]
    for k, e_ref in enumerate((e0_ref, e1_ref, e2_ref)):
        h = _gelu(ndb[:, k:k + 1] * w1s + b1_ref[...])
        e_ref[...] = _bdot(h, w2_ref[...]) + b2_ref[...]


def _layer_body(pn_ref, sf_ref, e0_ref, e1_ref, e2_ref,
                eW1a_ref, eW1b_ref, eW1c_ref, eb1_ref, eW2_ref, eb2_ref,
                nW1a_ref, nW1b_ref, nb1_ref, nW2_ref, nb2_ref,
                pno_ref, eo0_ref, eo1_ref, eo2_ref):
    pn = pn_ref[...]
    sf = sf_ref[...]
    rcon = _bdot(pn, eW1b_ref[...])
    nes = []
    for k, (e_ref, eo_ref) in enumerate(
            ((e0_ref, eo0_ref), (e1_ref, eo1_ref), (e2_ref, eo2_ref))):
        e = e_ref[...]
        x = (_bdot(sf[:, k * ND:(k + 1) * ND], eW1a_ref[...]) + rcon
             + _bdot(e, eW1c_ref[...]) + eb1_ref[...])
        ne = _bdot(_gelu(x), eW2_ref[...]) + eb2_ref[...]
        nes.append(ne)
        eo_ref[...] = e + ne
    agg = (nes[0] + nes[1]) + nes[2]
    nx = _bdot(pn, nW1a_ref[...]) + _bdot(agg, nW1b_ref[...]) + nb1_ref[...]
    nn = _bdot(_gelu(nx), nW2_ref[...]) + nb2_ref[...]
    pno_ref[...] = pn + nn


def _out_body(pn_ref, w1_ref, b1_ref, w2_ref, b2_ref, w3_ref, b3_ref, o_ref):
    h = _gelu(_bdot(pn_ref[...], w1_ref[...]) + b1_ref[...])
    h = _gelu(_bdot(h, w2_ref[...]) + b2_ref[...])
    o_ref[...] = _bdot(h, w3_ref[...]) + b3_ref[...]


def _full(shape):
    nd = len(shape)
    return pl.BlockSpec(shape, lambda i: (0,) * nd)


def _row(cols):
    return pl.BlockSpec((BD, cols), lambda i: (i, 0))


_GRID = (NQP // BD,)


def _pre(g, ndb, wts):
    return pl.pallas_call(
        _pre_body,
        grid=_GRID,
        in_specs=[_row(3 * ND), pl.BlockSpec((BD, KNN), lambda i: (i, 0))]
                 + [_full(w.shape) for w in wts],
        out_specs=[_row(ND), _row(ED), _row(ED), _row(ED)],
        out_shape=[jax.ShapeDtypeStruct((NQP, ND), jnp.float32)]
                  + [jax.ShapeDtypeStruct((NQP, ED), jnp.float32)] * 3,
    )(g, ndb, *wts)


def _layer(pn, sf, es, wts):
    return pl.pallas_call(
        _layer_body,
        grid=_GRID,
        in_specs=[_row(ND), _row(3 * ND), _row(ED), _row(ED), _row(ED)]
                 + [_full(w.shape) for w in wts],
        out_specs=[_row(ND), _row(ED), _row(ED), _row(ED)],
        out_shape=[jax.ShapeDtypeStruct((NQP, ND), jnp.float32)]
                  + [jax.ShapeDtypeStruct((NQP, ED), jnp.float32)] * 3,
    )(pn, sf, *es, *wts)


def _out(pn, wts):
    return pl.pallas_call(
        _out_body,
        grid=_GRID,
        in_specs=[_row(ND)] + [_full(w.shape) for w in wts],
        out_specs=pl.BlockSpec((BD, 3), lambda i: (i, 0)),
        out_shape=jax.ShapeDtypeStruct((NQP, 3), jnp.float32),
    )(pn, *wts)


def kernel(nodes, positions, query_positions, params):
    bf = jnp.bfloat16
    f32 = jnp.float32
    idx, nd = _knn(query_positions, positions)
    senders = idx.reshape(-1)

    encW1, encW2 = params["enc_W"]
    encb1, encb2 = params["enc_b"]
    w1s = jnp.sum(encW1.astype(bf).astype(f32), axis=0).reshape(1, ND)
    ndb = nd.astype(bf)
    pre_wts = (w1s, encb1.reshape(1, ND), encW2.astype(bf),
               encb2.reshape(1, ED))

    g0 = jnp.take(nodes, senders, axis=0).reshape(NQP, 3 * ND)
    pn, e0, e1, e2 = _pre(g0, ndb, pre_wts)

    for lp in params["layers"]:
        eW1, eW2 = lp["eW"]
        eb1, eb2 = lp["eb"]
        nW1, nW2 = lp["nW"]
        nb1, nb2 = lp["nb"]
        wts = (eW1[:ND].astype(bf), eW1[ND:2 * ND].astype(bf),
               eW1[2 * ND:].astype(bf), eb1.reshape(1, ND),
               eW2.astype(bf), eb2.reshape(1, ED),
               nW1[:ND].astype(bf), nW1[ND:].astype(bf), nb1.reshape(1, ND),
               nW2.astype(bf), nb2.reshape(1, ND))
        sf = jnp.take(pn, senders, axis=0).reshape(NQP, 3 * ND)
        pn, e0, e1, e2 = _layer(pn, sf, (e0, e1, e2), wts)

    oW1, oW2, oW3 = params["out_W"]
    ob1, ob2, ob3 = params["out_b"]
    out = _out(pn, (oW1.astype(bf), ob1.reshape(1, ND),
                    oW2.astype(bf), ob2.reshape(1, ND),
                    oW3.astype(bf), ob3.reshape(1, 3)))
    return out[:NQ]


# speedup vs baseline: 6.2331x; 1.3862x over previous
import functools

import jax
import jax.numpy as jnp
from jax import lax
from jax.experimental import pallas as pl
from jax.experimental.pallas import tpu as pltpu
from jax.experimental.pallas import tpu_sc as plsc

KNN = 3
NQ = 10000
NS = 10000
NQP = 10240
NSP = 10240
BQ = 256
BD = 512
ND = 128
ED = 16


def _knn_body(qb_ref, pt_ref, q2_ref, p2_ref, idx_ref, nd_ref):
    qb = qb_ref[...]
    pt = pt_ref[...]
    qp = jax.lax.dot_general(qb, pt, (((1,), (0,)), ((), ())),
                             preferred_element_type=jnp.float32)
    d2 = (q2_ref[...] + p2_ref[...]) - 2.0 * qp
    x = jnp.sqrt(jnp.maximum(d2, 1e-12))
    col = jax.lax.broadcasted_iota(jnp.int32, (BQ, NSP), 1)
    idxs, nds = [], []
    for _ in range(KNN):
        m = jnp.min(x, axis=1, keepdims=True)
        j = jnp.min(jnp.where(x == m, col, NSP), axis=1, keepdims=True)
        idxs.append(j)
        nds.append(m)
        x = jnp.where(col == j, jnp.float32(jnp.inf), x)
    idx_ref[...] = jnp.concatenate(idxs, axis=1)
    nd_ref[...] = jnp.concatenate(nds, axis=1)


def _knn(query_positions, positions):
    qb = jnp.pad(query_positions.astype(jnp.bfloat16),
                 ((0, NQP - NQ), (0, 5)))
    pt = jnp.pad(positions.astype(jnp.bfloat16),
                 ((0, NSP - NS), (0, 5))).T
    q2 = jnp.pad(jnp.sum(query_positions ** 2, axis=1),
                 (0, NQP - NQ)).reshape(NQP, 1)
    p2 = jnp.pad(jnp.sum(positions ** 2, axis=1),
                 (0, NSP - NS), constant_values=1e12).reshape(1, NSP)
    return pl.pallas_call(
        _knn_body,
        grid=(NQP // BQ,),
        in_specs=[
            pl.BlockSpec((BQ, 8), lambda i: (i, 0)),
            pl.BlockSpec((8, NSP), lambda i: (0, 0)),
            pl.BlockSpec((BQ, 1), lambda i: (i, 0)),
            pl.BlockSpec((1, NSP), lambda i: (0, 0)),
        ],
        out_specs=[
            pl.BlockSpec((BQ, KNN), lambda i: (i, 0)),
            pl.BlockSpec((BQ, KNN), lambda i: (i, 0)),
        ],
        out_shape=[
            jax.ShapeDtypeStruct((NQP, KNN), jnp.int32),
            jax.ShapeDtypeStruct((NQP, KNN), jnp.float32),
        ],
    )(qb, pt, q2, p2)


_NW = 32
_BPW = (3 * NQP) // _NW
_CHUNK = 120


def _sc_gather_body(table_hbm, idx_hbm, out_hbm, idx_v, rows_v, sem):
    wid = lax.axis_index("s") * 2 + lax.axis_index("c")
    base = wid * _BPW
    pltpu.sync_copy(idx_hbm.at[pl.ds(base, _BPW)], idx_v)
    copies = [
        pltpu.async_copy(
            table_hbm.at[idx_v.at[pl.ds(j * _CHUNK, _CHUNK)]],
            rows_v.at[pl.ds(j * _CHUNK, _CHUNK)], sem)
        for j in range(_BPW // _CHUNK)
    ]
    for c in copies:
        c.wait()
    pltpu.sync_copy(rows_v, out_hbm.at[pl.ds(base, _BPW)])


@functools.partial(
    pl.kernel,
    mesh=plsc.VectorSubcoreMesh(core_axis_name="c", subcore_axis_name="s"),
    out_type=jax.ShapeDtypeStruct((3 * NQP, ND), jnp.float32),
    scratch_types=[
        pltpu.VMEM((_BPW,), jnp.int32),
        pltpu.VMEM((_BPW, ND), jnp.float32),
        pltpu.SemaphoreType.DMA,
    ],
)
def _sc_gather(table_hbm, idx_hbm, out_hbm, idx_v, rows_v, sem):
    _sc_gather_body(table_hbm, idx_hbm, out_hbm, idx_v, rows_v, sem)


_SQRT_HALF = 0.7071067811865476


def _gelu(x):
    return 0.5 * x * (1.0 + jax.lax.erf(x * _SQRT_HALF))


def _bdot(a, b):
    return jax.lax.dot_general(a.astype(jnp.bfloat16), b,
                               (((1,), (0,)), ((), ())),
                               preferred_element_type=jnp.float32)


def _pre_body(g_ref, ndb_ref, w1s_ref, b1_ref, w2_ref, b2_ref,
              pn_ref, e0_ref, e1_ref, e2_ref):
    g = g_ref[...]
    s = (g[:, :ND] + g[:, ND:2 * ND]) + g[:, 2 * ND:3 * ND]
    pn_ref[...] = s / 3.0
    ndb = ndb_ref[...].astype(jnp.float32)
    w1s = w1s_ref[...]
    for k, e_ref in enumerate((e0_ref, e1_ref, e2_ref)):
        h = _gelu(ndb[:, k:k + 1] * w1s + b1_ref[...])
        e_ref[...] = _bdot(h, w2_ref[...]) + b2_ref[...]


def _layer_body(pn_ref, sf_ref, e0_ref, e1_ref, e2_ref,
                eW1a_ref, eW1b_ref, eW1c_ref, eb1_ref, eW2_ref, eb2_ref,
                nW1a_ref, nW1b_ref, nb1_ref, nW2_ref, nb2_ref,
                pno_ref, eo0_ref, eo1_ref, eo2_ref):
    pn = pn_ref[...]
    sf = sf_ref[...]
    rcon = _bdot(pn, eW1b_ref[...])
    nes = []
    for k, (e_ref, eo_ref) in enumerate(
            ((e0_ref, eo0_ref), (e1_ref, eo1_ref), (e2_ref, eo2_ref))):
        e = e_ref[...]
        x = (_bdot(sf[:, k * ND:(k + 1) * ND], eW1a_ref[...]) + rcon
             + _bdot(e, eW1c_ref[...]) + eb1_ref[...])
        ne = _bdot(_gelu(x), eW2_ref[...]) + eb2_ref[...]
        nes.append(ne)
        eo_ref[...] = e + ne
    agg = (nes[0] + nes[1]) + nes[2]
    nx = _bdot(pn, nW1a_ref[...]) + _bdot(agg, nW1b_ref[...]) + nb1_ref[...]
    nn = _bdot(_gelu(nx), nW2_ref[...]) + nb2_ref[...]
    pno_ref[...] = pn + nn


def _out_body(pn_ref, w1_ref, b1_ref, w2_ref, b2_ref, w3_ref, b3_ref, o_ref):
    h = _gelu(_bdot(pn_ref[...], w1_ref[...]) + b1_ref[...])
    h = _gelu(_bdot(h, w2_ref[...]) + b2_ref[...])
    o_ref[...] = _bdot(h, w3_ref[...]) + b3_ref[...]


def _full(shape):
    nd = len(shape)
    return pl.BlockSpec(shape, lambda i: (0,) * nd)


def _row(cols):
    return pl.BlockSpec((BD, cols), lambda i: (i, 0))


_GRID = (NQP // BD,)


def _pre(g, ndb, wts):
    return pl.pallas_call(
        _pre_body,
        grid=_GRID,
        in_specs=[_row(3 * ND), pl.BlockSpec((BD, KNN), lambda i: (i, 0))]
                 + [_full(w.shape) for w in wts],
        out_specs=[_row(ND), _row(ED), _row(ED), _row(ED)],
        out_shape=[jax.ShapeDtypeStruct((NQP, ND), jnp.float32)]
                  + [jax.ShapeDtypeStruct((NQP, ED), jnp.float32)] * 3,
    )(g, ndb, *wts)


def _layer(pn, sf, es, wts):
    return pl.pallas_call(
        _layer_body,
        grid=_GRID,
        in_specs=[_row(ND), _row(3 * ND), _row(ED), _row(ED), _row(ED)]
                 + [_full(w.shape) for w in wts],
        out_specs=[_row(ND), _row(ED), _row(ED), _row(ED)],
        out_shape=[jax.ShapeDtypeStruct((NQP, ND), jnp.float32)]
                  + [jax.ShapeDtypeStruct((NQP, ED), jnp.float32)] * 3,
    )(pn, sf, *es, *wts)


def _out(pn, wts):
    return pl.pallas_call(
        _out_body,
        grid=_GRID,
        in_specs=[_row(ND)] + [_full(w.shape) for w in wts],
        out_specs=pl.BlockSpec((BD, 3), lambda i: (i, 0)),
        out_shape=jax.ShapeDtypeStruct((NQP, 3), jnp.float32),
    )(pn, *wts)


def kernel(nodes, positions, query_positions, params):
    bf = jnp.bfloat16
    f32 = jnp.float32
    idx, nd = _knn(query_positions, positions)
    senders = idx.reshape(-1)

    encW1, encW2 = params["enc_W"]
    encb1, encb2 = params["enc_b"]
    w1s = jnp.sum(encW1.astype(bf).astype(f32), axis=0).reshape(1, ND)
    ndb = nd.astype(bf)
    pre_wts = (w1s, encb1.reshape(1, ND), encW2.astype(bf),
               encb2.reshape(1, ED))

    g0 = _sc_gather(nodes, senders).reshape(NQP, 3 * ND)
    pn, e0, e1, e2 = _pre(g0, ndb, pre_wts)

    for lp in params["layers"]:
        eW1, eW2 = lp["eW"]
        eb1, eb2 = lp["eb"]
        nW1, nW2 = lp["nW"]
        nb1, nb2 = lp["nb"]
        wts = (eW1[:ND].astype(bf), eW1[ND:2 * ND].astype(bf),
               eW1[2 * ND:].astype(bf), eb1.reshape(1, ND),
               eW2.astype(bf), eb2.reshape(1, ED),
               nW1[:ND].astype(bf), nW1[ND:].astype(bf), nb1.reshape(1, ND),
               nW2.astype(bf), nb2.reshape(1, ND))
        sf = _sc_gather(pn, senders).reshape(NQP, 3 * ND)
        pn, e0, e1, e2 = _layer(pn, sf, (e0, e1, e2), wts)

    oW1, oW2, oW3 = params["out_W"]
    ob1, ob2, ob3 = params["out_b"]
    out = _out(pn, (oW1.astype(bf), ob1.reshape(1, ND),
                    oW2.astype(bf), ob2.reshape(1, ND),
                    oW3.astype(bf), ob3.reshape(1, 3)))
    return out[:NQ]


# speedup vs baseline: 6.8214x; 1.0944x over previous
import functools

import jax
import jax.numpy as jnp
from jax import lax
from jax.experimental import pallas as pl
from jax.experimental.pallas import tpu as pltpu
from jax.experimental.pallas import tpu_sc as plsc

KNN = 3
NQ = 10000
NS = 10000
NQP = 10240
NSP = 10240
BQ = 256
BD = 512
ND = 128
ED = 16


def _knn_body(qb_ref, pt2_ref, q2_ref, p2_ref, idx_ref, nd_ref):
    qb = qb_ref[...]
    pt2 = pt2_ref[...]
    qp2 = jax.lax.dot_general(qb, pt2, (((1,), (0,)), ((), ())),
                              preferred_element_type=jnp.float32)
    x = (q2_ref[...] + p2_ref[...]) - qp2
    col = jax.lax.broadcasted_iota(jnp.int32, (BQ, NSP), 1)
    idxs, nds = [], []
    for k in range(KNN):
        m = jnp.min(x, axis=1, keepdims=True)
        j = jnp.min(jnp.where(x == m, col, NSP), axis=1, keepdims=True)
        idxs.append(j)
        nds.append(jnp.sqrt(jnp.maximum(m, 1e-12)))
        if k + 1 < KNN:
            x = jnp.where(col == j, jnp.float32(jnp.inf), x)
    idx_ref[...] = jnp.concatenate(idxs, axis=1)
    nd_ref[...] = jnp.concatenate(nds, axis=1)


def _knn(query_positions, positions):
    qb = jnp.pad(query_positions.astype(jnp.bfloat16),
                 ((0, NQP - NQ), (0, 5)))
    pt2 = jnp.pad(positions.astype(jnp.bfloat16) * 2,
                  ((0, NSP - NS), (0, 5))).T
    q2 = jnp.pad(jnp.sum(query_positions ** 2, axis=1),
                 (0, NQP - NQ)).reshape(NQP, 1)
    p2 = jnp.pad(jnp.sum(positions ** 2, axis=1),
                 (0, NSP - NS), constant_values=1e12).reshape(1, NSP)
    return pl.pallas_call(
        _knn_body,
        grid=(NQP // BQ,),
        in_specs=[
            pl.BlockSpec((BQ, 8), lambda i: (i, 0)),
            pl.BlockSpec((8, NSP), lambda i: (0, 0)),
            pl.BlockSpec((BQ, 1), lambda i: (i, 0)),
            pl.BlockSpec((1, NSP), lambda i: (0, 0)),
        ],
        out_specs=[
            pl.BlockSpec((BQ, KNN), lambda i: (i, 0)),
            pl.BlockSpec((BQ, KNN), lambda i: (i, 0)),
        ],
        out_shape=[
            jax.ShapeDtypeStruct((NQP, KNN), jnp.int32),
            jax.ShapeDtypeStruct((NQP, KNN), jnp.float32),
        ],
    )(qb, pt2, q2, p2)


_NW = 32
_BPW = NQP // _NW
_CHUNK = 80


@functools.partial(
    pl.kernel,
    mesh=plsc.VectorSubcoreMesh(core_axis_name="c", subcore_axis_name="s"),
    out_type=[jax.ShapeDtypeStruct((NQP, ND), jnp.float32)] * KNN,
    scratch_types=[
        pltpu.VMEM((_BPW,), jnp.int32),
        pltpu.VMEM((_BPW, ND), jnp.float32),
        pltpu.SemaphoreType.DMA,
    ],
)
def _sc_gather(table_hbm, i0_hbm, i1_hbm, i2_hbm,
               o0_hbm, o1_hbm, o2_hbm, idx_v, rows_v, sem):
    wid = lax.axis_index("s") * 2 + lax.axis_index("c")
    base = wid * _BPW
    for idx_hbm, out_hbm in ((i0_hbm, o0_hbm), (i1_hbm, o1_hbm),
                             (i2_hbm, o2_hbm)):
        pltpu.sync_copy(idx_hbm.at[pl.ds(base, _BPW)], idx_v)
        copies = [
            pltpu.async_copy(
                table_hbm.at[idx_v.at[pl.ds(j * _CHUNK, _CHUNK)]],
                rows_v.at[pl.ds(j * _CHUNK, _CHUNK)], sem)
            for j in range(_BPW // _CHUNK)
        ]
        for c in copies:
            c.wait()
        pltpu.sync_copy(rows_v, out_hbm.at[pl.ds(base, _BPW)])


_SQRT_HALF = 0.7071067811865476


def _gelu(x):
    return 0.5 * x * (1.0 + jax.lax.erf(x * _SQRT_HALF))


def _bdot(a, b):
    return jax.lax.dot_general(a.astype(jnp.bfloat16), b,
                               (((1,), (0,)), ((), ())),
                               preferred_element_type=jnp.float32)


def _pre_body(g0_ref, g1_ref, g2_ref, ndb_ref, w1s_ref, b1_ref, w2_ref,
              b2_ref, pn_ref, e0_ref, e1_ref, e2_ref):
    s = (g0_ref[...] + g1_ref[...]) + g2_ref[...]
    pn_ref[...] = s / 3.0
    ndb = ndb_ref[...].astype(jnp.float32)
    w1s = w1s_ref[...]
    for k, e_ref in enumerate((e0_ref, e1_ref, e2_ref)):
        h = _gelu(ndb[:, k:k + 1] * w1s + b1_ref[...])
        e_ref[...] = _bdot(h, w2_ref[...]) + b2_ref[...]


def _layer_core(pn, sfs, es,
                eW1a, eW1b, eW1c, eb1, eW2, eb2, nW1a, nW1b, nb1, nW2, nb2):
    rcon = _bdot(pn, eW1b)
    nes = []
    for sf, e in zip(sfs, es):
        xk = _bdot(sf, eW1a) + rcon + _bdot(e, eW1c) + eb1
        nes.append(_bdot(_gelu(xk), eW2) + eb2)
    agg = (nes[0] + nes[1]) + nes[2]
    nx = _bdot(pn, nW1a) + _bdot(agg, nW1b) + nb1
    nn = _bdot(_gelu(nx), nW2) + nb2
    return pn + nn, nes


def _layer_body(pn_ref, s0_ref, s1_ref, s2_ref, e0_ref, e1_ref, e2_ref,
                eW1a_ref, eW1b_ref, eW1c_ref, eb1_ref, eW2_ref, eb2_ref,
                nW1a_ref, nW1b_ref, nb1_ref, nW2_ref, nb2_ref,
                pno_ref, eo0_ref, eo1_ref, eo2_ref):
    es = (e0_ref[...], e1_ref[...], e2_ref[...])
    pno, nes = _layer_core(
        pn_ref[...], (s0_ref[...], s1_ref[...], s2_ref[...]), es,
        eW1a_ref[...], eW1b_ref[...], eW1c_ref[...], eb1_ref[...],
        eW2_ref[...], eb2_ref[...], nW1a_ref[...], nW1b_ref[...],
        nb1_ref[...], nW2_ref[...], nb2_ref[...])
    pno_ref[...] = pno
    for e, ne, eo_ref in zip(es, nes, (eo0_ref, eo1_ref, eo2_ref)):
        eo_ref[...] = e + ne


def _layer_out_body(pn_ref, s0_ref, s1_ref, s2_ref, e0_ref, e1_ref, e2_ref,
                    eW1a_ref, eW1b_ref, eW1c_ref, eb1_ref, eW2_ref, eb2_ref,
                    nW1a_ref, nW1b_ref, nb1_ref, nW2_ref, nb2_ref,
                    oW1_ref, ob1_ref, oW2_ref, ob2_ref, oW3_ref, ob3_ref,
                    o_ref):
    pno, _ = _layer_core(
        pn_ref[...], (s0_ref[...], s1_ref[...], s2_ref[...]),
        (e0_ref[...], e1_ref[...], e2_ref[...]),
        eW1a_ref[...], eW1b_ref[...], eW1c_ref[...], eb1_ref[...],
        eW2_ref[...], eb2_ref[...], nW1a_ref[...], nW1b_ref[...],
        nb1_ref[...], nW2_ref[...], nb2_ref[...])
    h = _gelu(_bdot(pno, oW1_ref[...]) + ob1_ref[...])
    h = _gelu(_bdot(h, oW2_ref[...]) + ob2_ref[...])
    o_ref[...] = _bdot(h, oW3_ref[...]) + ob3_ref[...]


def _full(shape):
    nd = len(shape)
    return pl.BlockSpec(shape, lambda i: (0,) * nd)


def _row(cols):
    return pl.BlockSpec((BD, cols), lambda i: (i, 0))


_GRID = (NQP // BD,)


def _pre(gs, ndb, wts):
    return pl.pallas_call(
        _pre_body,
        grid=_GRID,
        in_specs=[_row(ND)] * 3 + [pl.BlockSpec((BD, KNN), lambda i: (i, 0))]
                 + [_full(w.shape) for w in wts],
        out_specs=[_row(ND), _row(ED), _row(ED), _row(ED)],
        out_shape=[jax.ShapeDtypeStruct((NQP, ND), jnp.float32)]
                  + [jax.ShapeDtypeStruct((NQP, ED), jnp.float32)] * 3,
    )(*gs, ndb, *wts)


def _layer(pn, sfs, es, wts):
    return pl.pallas_call(
        _layer_body,
        grid=_GRID,
        in_specs=[_row(ND)] * 4 + [_row(ED)] * 3
                 + [_full(w.shape) for w in wts],
        out_specs=[_row(ND), _row(ED), _row(ED), _row(ED)],
        out_shape=[jax.ShapeDtypeStruct((NQP, ND), jnp.float32)]
                  + [jax.ShapeDtypeStruct((NQP, ED), jnp.float32)] * 3,
    )(pn, *sfs, *es, *wts)


def _layer_out(pn, sfs, es, wts, owts):
    return pl.pallas_call(
        _layer_out_body,
        grid=_GRID,
        in_specs=[_row(ND)] * 4 + [_row(ED)] * 3
                 + [_full(w.shape) for w in wts]
                 + [_full(w.shape) for w in owts],
        out_specs=pl.BlockSpec((BD, 3), lambda i: (i, 0)),
        out_shape=jax.ShapeDtypeStruct((NQP, 3), jnp.float32),
    )(pn, *sfs, *es, *wts, *owts)


def _layer_wts(lp, bf):
    eW1, eW2 = lp["eW"]
    eb1, eb2 = lp["eb"]
    nW1, nW2 = lp["nW"]
    nb1, nb2 = lp["nb"]
    return (eW1[:ND].astype(bf), eW1[ND:2 * ND].astype(bf),
            eW1[2 * ND:].astype(bf), eb1.reshape(1, ND),
            eW2.astype(bf), eb2.reshape(1, ED),
            nW1[:ND].astype(bf), nW1[ND:].astype(bf), nb1.reshape(1, ND),
            nW2.astype(bf), nb2.reshape(1, ND))


def kernel(nodes, positions, query_positions, params):
    bf = jnp.bfloat16
    f32 = jnp.float32
    idx, nd = _knn(query_positions, positions)
    senders = tuple(idx[:, k] for k in range(KNN))

    encW1, encW2 = params["enc_W"]
    encb1, encb2 = params["enc_b"]
    w1s = jnp.sum(encW1.astype(bf).astype(f32), axis=0).reshape(1, ND)
    ndb = nd.astype(bf)
    pre_wts = (w1s, encb1.reshape(1, ND), encW2.astype(bf),
               encb2.reshape(1, ED))

    gs = _sc_gather(nodes, *senders)
    pn, e0, e1, e2 = _pre(gs, ndb, pre_wts)

    lps = params["layers"]
    sfs = _sc_gather(pn, *senders)
    pn, e0, e1, e2 = _layer(pn, sfs, (e0, e1, e2), _layer_wts(lps[0], bf))

    oW1, oW2, oW3 = params["out_W"]
    ob1, ob2, ob3 = params["out_b"]
    owts = (oW1.astype(bf), ob1.reshape(1, ND),
            oW2.astype(bf), ob2.reshape(1, ND),
            oW3.astype(bf), ob3.reshape(1, 3))
    sfs = _sc_gather(pn, *senders)
    out = _layer_out(pn, sfs, (e0, e1, e2), _layer_wts(lps[1], bf), owts)
    return out[:NQ]


# speedup vs baseline: 7.1532x; 1.0486x over previous
import functools

import jax
import jax.numpy as jnp
from jax import lax
from jax.experimental import pallas as pl
from jax.experimental.pallas import tpu as pltpu
from jax.experimental.pallas import tpu_sc as plsc

KNN = 3
NQ = 10000
NS = 10000
NQP = 10240
NSP = 10240
BQ = 256
BD = 512
ND = 128
ED = 16


def _knn_body(qb_ref, pt2_ref, q2_ref, p2_ref, idx_ref, nd_ref):
    qb = qb_ref[...]
    pt2 = pt2_ref[...]
    qp2 = jax.lax.dot_general(qb, pt2, (((1,), (0,)), ((), ())),
                              preferred_element_type=jnp.float32)
    d2 = (q2_ref[...] + p2_ref[...]) - qp2
    x = jnp.maximum(d2, 1e-12)
    col = jax.lax.broadcasted_iota(jnp.int32, (BQ, NSP), 1)
    idxs, nds = [], []
    for k in range(KNN):
        m = jnp.min(x, axis=1, keepdims=True)
        j = jnp.min(jnp.where(x == m, col, NSP), axis=1, keepdims=True)
        idxs.append(j)
        nds.append(jnp.sqrt(m))
        if k + 1 < KNN:
            x = jnp.where(col == j, jnp.float32(jnp.inf), x)
    idx_ref[...] = jnp.concatenate(idxs, axis=1)
    nd_ref[...] = jnp.concatenate(nds, axis=1)


def _knn(query_positions, positions):
    qb = jnp.pad(query_positions.astype(jnp.bfloat16),
                 ((0, NQP - NQ), (0, 5)))
    pt2 = jnp.pad(positions.astype(jnp.bfloat16) * 2,
                  ((0, NSP - NS), (0, 5))).T
    q2 = jnp.pad(jnp.sum(query_positions ** 2, axis=1),
                 (0, NQP - NQ)).reshape(NQP, 1)
    p2 = jnp.pad(jnp.sum(positions ** 2, axis=1),
                 (0, NSP - NS), constant_values=1e12).reshape(1, NSP)
    return pl.pallas_call(
        _knn_body,
        grid=(NQP // BQ,),
        in_specs=[
            pl.BlockSpec((BQ, 8), lambda i: (i, 0)),
            pl.BlockSpec((8, NSP), lambda i: (0, 0)),
            pl.BlockSpec((BQ, 1), lambda i: (i, 0)),
            pl.BlockSpec((1, NSP), lambda i: (0, 0)),
        ],
        out_specs=[
            pl.BlockSpec((BQ, KNN), lambda i: (i, 0)),
            pl.BlockSpec((BQ, KNN), lambda i: (i, 0)),
        ],
        out_shape=[
            jax.ShapeDtypeStruct((NQP, KNN), jnp.int32),
            jax.ShapeDtypeStruct((NQP, KNN), jnp.float32),
        ],
    )(qb, pt2, q2, p2)


_NW = 32
_BPW = NQP // _NW
_CHUNK = 80


@functools.partial(
    pl.kernel,
    mesh=plsc.VectorSubcoreMesh(core_axis_name="c", subcore_axis_name="s"),
    out_type=[jax.ShapeDtypeStruct((NQP, ND), jnp.float32)] * KNN,
    scratch_types=[
        pltpu.VMEM((_BPW,), jnp.int32),
        pltpu.VMEM((_BPW, ND), jnp.float32),
        pltpu.SemaphoreType.DMA,
    ],
)
def _sc_gather(table_hbm, i0_hbm, i1_hbm, i2_hbm,
               o0_hbm, o1_hbm, o2_hbm, idx_v, rows_v, sem):
    wid = lax.axis_index("s") * 2 + lax.axis_index("c")
    base = wid * _BPW
    for idx_hbm, out_hbm in ((i0_hbm, o0_hbm), (i1_hbm, o1_hbm),
                             (i2_hbm, o2_hbm)):
        pltpu.sync_copy(idx_hbm.at[pl.ds(base, _BPW)], idx_v)
        copies = [
            pltpu.async_copy(
                table_hbm.at[idx_v.at[pl.ds(j * _CHUNK, _CHUNK)]],
                rows_v.at[pl.ds(j * _CHUNK, _CHUNK)], sem)
            for j in range(_BPW // _CHUNK)
        ]
        for c in copies:
            c.wait()
        pltpu.sync_copy(rows_v, out_hbm.at[pl.ds(base, _BPW)])


_SQRT_HALF = 0.7071067811865476


def _gelu(x):
    return 0.5 * x * (1.0 + jax.lax.erf(x * _SQRT_HALF))


def _bdot(a, b):
    return jax.lax.dot_general(a.astype(jnp.bfloat16), b,
                               (((1,), (0,)), ((), ())),
                               preferred_element_type=jnp.float32)


def _pre_body(g0_ref, g1_ref, g2_ref, ndb_ref, w1s_ref, b1_ref, w2_ref,
              b2_ref, pn_ref, e0_ref, e1_ref, e2_ref):
    s = (g0_ref[...] + g1_ref[...]) + g2_ref[...]
    pn_ref[...] = s / 3.0
    ndb = ndb_ref[...].astype(jnp.float32)
    w1s = w1s_ref[...]
    for k, e_ref in enumerate((e0_ref, e1_ref, e2_ref)):
        h = _gelu(ndb[:, k:k + 1] * w1s + b1_ref[...])
        e_ref[...] = _bdot(h, w2_ref[...]) + b2_ref[...]


def _layer_core(pn, sfs, es,
                eW1a, eW1b, eW1c, eb1, eW2, eb2, nW1a, nW1b, nb1, nW2, nb2):
    rcon = _bdot(pn, eW1b)
    nes = []
    for sf, e in zip(sfs, es):
        xk = _bdot(sf, eW1a) + rcon + _bdot(e, eW1c) + eb1
        nes.append(_bdot(_gelu(xk), eW2) + eb2)
    agg = (nes[0] + nes[1]) + nes[2]
    nx = _bdot(pn, nW1a) + _bdot(agg, nW1b) + nb1
    nn = _bdot(_gelu(nx), nW2) + nb2
    return pn + nn, nes


def _layer_body(pn_ref, s0_ref, s1_ref, s2_ref, e0_ref, e1_ref, e2_ref,
                eW1a_ref, eW1b_ref, eW1c_ref, eb1_ref, eW2_ref, eb2_ref,
                nW1a_ref, nW1b_ref, nb1_ref, nW2_ref, nb2_ref,
                pno_ref, eo0_ref, eo1_ref, eo2_ref):
    es = (e0_ref[...], e1_ref[...], e2_ref[...])
    pno, nes = _layer_core(
        pn_ref[...], (s0_ref[...], s1_ref[...], s2_ref[...]), es,
        eW1a_ref[...], eW1b_ref[...], eW1c_ref[...], eb1_ref[...],
        eW2_ref[...], eb2_ref[...], nW1a_ref[...], nW1b_ref[...],
        nb1_ref[...], nW2_ref[...], nb2_ref[...])
    pno_ref[...] = pno
    for e, ne, eo_ref in zip(es, nes, (eo0_ref, eo1_ref, eo2_ref)):
        eo_ref[...] = e + ne


def _layer_out_body(pn_ref, s0_ref, s1_ref, s2_ref, e0_ref, e1_ref, e2_ref,
                    eW1a_ref, eW1b_ref, eW1c_ref, eb1_ref, eW2_ref, eb2_ref,
                    nW1a_ref, nW1b_ref, nb1_ref, nW2_ref, nb2_ref,
                    oW1_ref, ob1_ref, oW2_ref, ob2_ref, oW3_ref, ob3_ref,
                    o_ref):
    pno, _ = _layer_core(
        pn_ref[...], (s0_ref[...], s1_ref[...], s2_ref[...]),
        (e0_ref[...], e1_ref[...], e2_ref[...]),
        eW1a_ref[...], eW1b_ref[...], eW1c_ref[...], eb1_ref[...],
        eW2_ref[...], eb2_ref[...], nW1a_ref[...], nW1b_ref[...],
        nb1_ref[...], nW2_ref[...], nb2_ref[...])
    h = _gelu(_bdot(pno, oW1_ref[...]) + ob1_ref[...])
    h = _gelu(_bdot(h, oW2_ref[...]) + ob2_ref[...])
    o_ref[...] = _bdot(h, oW3_ref[...]) + ob3_ref[...]


def _full(shape):
    nd = len(shape)
    return pl.BlockSpec(shape, lambda i: (0,) * nd)


def _row(cols):
    return pl.BlockSpec((BD, cols), lambda i: (i, 0))


_GRID = (NQP // BD,)


def _pre(gs, ndb, wts):
    return pl.pallas_call(
        _pre_body,
        grid=_GRID,
        in_specs=[_row(ND)] * 3 + [pl.BlockSpec((BD, KNN), lambda i: (i, 0))]
                 + [_full(w.shape) for w in wts],
        out_specs=[_row(ND), _row(ED), _row(ED), _row(ED)],
        out_shape=[jax.ShapeDtypeStruct((NQP, ND), jnp.float32)]
                  + [jax.ShapeDtypeStruct((NQP, ED), jnp.float32)] * 3,
    )(*gs, ndb, *wts)


def _layer(pn, sfs, es, wts):
    return pl.pallas_call(
        _layer_body,
        grid=_GRID,
        in_specs=[_row(ND)] * 4 + [_row(ED)] * 3
                 + [_full(w.shape) for w in wts],
        out_specs=[_row(ND), _row(ED), _row(ED), _row(ED)],
        out_shape=[jax.ShapeDtypeStruct((NQP, ND), jnp.float32)]
                  + [jax.ShapeDtypeStruct((NQP, ED), jnp.float32)] * 3,
    )(pn, *sfs, *es, *wts)


def _layer_out(pn, sfs, es, wts, owts):
    return pl.pallas_call(
        _layer_out_body,
        grid=_GRID,
        in_specs=[_row(ND)] * 4 + [_row(ED)] * 3
                 + [_full(w.shape) for w in wts]
                 + [_full(w.shape) for w in owts],
        out_specs=pl.BlockSpec((BD, 3), lambda i: (i, 0)),
        out_shape=jax.ShapeDtypeStruct((NQP, 3), jnp.float32),
    )(pn, *sfs, *es, *wts, *owts)


def _layer_wts(lp, bf):
    eW1, eW2 = lp["eW"]
    eb1, eb2 = lp["eb"]
    nW1, nW2 = lp["nW"]
    nb1, nb2 = lp["nb"]
    return (eW1[:ND].astype(bf), eW1[ND:2 * ND].astype(bf),
            eW1[2 * ND:].astype(bf), eb1.reshape(1, ND),
            eW2.astype(bf), eb2.reshape(1, ED),
            nW1[:ND].astype(bf), nW1[ND:].astype(bf), nb1.reshape(1, ND),
            nW2.astype(bf), nb2.reshape(1, ND))


def kernel(nodes, positions, query_positions, params):
    bf = jnp.bfloat16
    f32 = jnp.float32
    idx, nd = _knn(query_positions, positions)
    senders = tuple(idx[:, k] for k in range(KNN))

    encW1, encW2 = params["enc_W"]
    encb1, encb2 = params["enc_b"]
    w1s = jnp.sum(encW1.astype(bf).astype(f32), axis=0).reshape(1, ND)
    ndb = nd.astype(bf)
    pre_wts = (w1s, encb1.reshape(1, ND), encW2.astype(bf),
               encb2.reshape(1, ED))

    gs = _sc_gather(nodes, *senders)
    pn, e0, e1, e2 = _pre(gs, ndb, pre_wts)

    lps = params["layers"]
    sfs = _sc_gather(pn, *senders)
    pn, e0, e1, e2 = _layer(pn, sfs, (e0, e1, e2), _layer_wts(lps[0], bf))

    oW1, oW2, oW3 = params["out_W"]
    ob1, ob2, ob3 = params["out_b"]
    owts = (oW1.astype(bf), ob1.reshape(1, ND),
            oW2.astype(bf), ob2.reshape(1, ND),
            oW3.astype(bf), ob3.reshape(1, 3))
    sfs = _sc_gather(pn, *senders)
    out = _layer_out(pn, sfs, (e0, e1, e2), _layer_wts(lps[1], bf), owts)
    return out[:NQ]


# speedup vs baseline: 8.2627x; 1.1551x over previous
import functools

import jax
import jax.numpy as jnp
from jax import lax
from jax.experimental import pallas as pl
from jax.experimental.pallas import tpu as pltpu
from jax.experimental.pallas import tpu_sc as plsc

KNN = 3
NQ = 10000
NS = 10000
NQP = 10240
NSP = 10240
BQ = 256
BD = 512
ND = 128
ED = 16


def _knn_body(qb_ref, pt2_ref, q2_ref, p2_ref, idx_ref, nd_ref):
    qb = qb_ref[...]
    pt2 = pt2_ref[...]
    qp2 = jax.lax.dot_general(qb, pt2, (((1,), (0,)), ((), ())),
                              preferred_element_type=jnp.float32)
    d2 = (q2_ref[...] + p2_ref[...]) - qp2
    x = jnp.maximum(d2, 1e-12)
    col = jax.lax.broadcasted_iota(jnp.int32, (BQ, NSP), 1)
    idxs, nds = [], []
    for k in range(KNN):
        m = jnp.min(x, axis=1, keepdims=True)
        j = jnp.min(jnp.where(x == m, col, NSP), axis=1, keepdims=True)
        idxs.append(j)
        nds.append(jnp.sqrt(m))
        if k + 1 < KNN:
            x = jnp.where(col == j, jnp.float32(jnp.inf), x)
    idx_ref[...] = jnp.concatenate(idxs, axis=1)
    nd_ref[...] = jnp.concatenate(nds, axis=1)


def _knn(query_positions, positions):
    qb = jnp.pad(query_positions.astype(jnp.bfloat16),
                 ((0, NQP - NQ), (0, 5)))
    pt2 = jnp.pad(positions.astype(jnp.bfloat16) * 2,
                  ((0, NSP - NS), (0, 5))).T
    q2 = jnp.pad(jnp.sum(query_positions ** 2, axis=1),
                 (0, NQP - NQ)).reshape(NQP, 1)
    p2 = jnp.pad(jnp.sum(positions ** 2, axis=1),
                 (0, NSP - NS), constant_values=1e12).reshape(1, NSP)
    return pl.pallas_call(
        _knn_body,
        grid=(NQP // BQ,),
        in_specs=[
            pl.BlockSpec((BQ, 8), lambda i: (i, 0)),
            pl.BlockSpec((8, NSP), lambda i: (0, 0)),
            pl.BlockSpec((BQ, 1), lambda i: (i, 0)),
            pl.BlockSpec((1, NSP), lambda i: (0, 0)),
        ],
        out_specs=[
            pl.BlockSpec((BQ, KNN), lambda i: (i, 0)),
            pl.BlockSpec((BQ, KNN), lambda i: (i, 0)),
        ],
        out_shape=[
            jax.ShapeDtypeStruct((NQP, KNN), jnp.int32),
            jax.ShapeDtypeStruct((NQP, KNN), jnp.float32),
        ],
    )(qb, pt2, q2, p2)


_NW = 32
_BPW = NQP // _NW
_CHUNK = 80


@functools.partial(
    pl.kernel,
    mesh=plsc.VectorSubcoreMesh(core_axis_name="c", subcore_axis_name="s"),
    out_type=[jax.ShapeDtypeStruct((NQP, ND), jnp.float32)] * KNN,
    scratch_types=[
        pltpu.VMEM((KNN * _BPW,), jnp.int32),
        pltpu.VMEM((KNN * _BPW, ND), jnp.float32),
        pltpu.SemaphoreType.DMA,
        pltpu.SemaphoreType.DMA,
        pltpu.SemaphoreType.DMA,
        pltpu.SemaphoreType.DMA,
    ],
)
def _sc_gather(table_hbm, i0_hbm, i1_hbm, i2_hbm,
               o0_hbm, o1_hbm, o2_hbm, idx_v, rows_v, sem_i, s0, s1, s2):
    wid = lax.axis_index("s") * 2 + lax.axis_index("c")
    base = wid * _BPW
    sems = (s0, s1, s2)
    icopies = [
        pltpu.async_copy(idx_hbm.at[pl.ds(base, _BPW)],
                         idx_v.at[pl.ds(k * _BPW, _BPW)], sem_i)
        for k, idx_hbm in enumerate((i0_hbm, i1_hbm, i2_hbm))
    ]
    for c in icopies:
        c.wait()
    copies = [
        [pltpu.async_copy(
            table_hbm.at[idx_v.at[pl.ds(k * _BPW + j * _CHUNK, _CHUNK)]],
            rows_v.at[pl.ds(k * _BPW + j * _CHUNK, _CHUNK)], sems[k])
         for j in range(_BPW // _CHUNK)]
        for k in range(KNN)
    ]
    for k, out_hbm in enumerate((o0_hbm, o1_hbm, o2_hbm)):
        for c in copies[k]:
            c.wait()
        pltpu.sync_copy(rows_v.at[pl.ds(k * _BPW, _BPW)],
                        out_hbm.at[pl.ds(base, _BPW)])


_SQRT_HALF = 0.7071067811865476


def _gelu(x):
    return 0.5 * x * (1.0 + jax.lax.erf(x * _SQRT_HALF))


def _bdot(a, b):
    return jax.lax.dot_general(a.astype(jnp.bfloat16), b,
                               (((1,), (0,)), ((), ())),
                               preferred_element_type=jnp.float32)


def _pre_body(g0_ref, g1_ref, g2_ref, ndb_ref, w1s_ref, b1_ref, w2_ref,
              b2_ref, pn_ref, e0_ref, e1_ref, e2_ref):
    s = (g0_ref[...] + g1_ref[...]) + g2_ref[...]
    pn_ref[...] = s / 3.0
    ndb = ndb_ref[...].astype(jnp.float32)
    w1s = w1s_ref[...]
    for k, e_ref in enumerate((e0_ref, e1_ref, e2_ref)):
        h = _gelu(ndb[:, k:k + 1] * w1s + b1_ref[...])
        e_ref[...] = _bdot(h, w2_ref[...]) + b2_ref[...]


def _layer_core(pn, sfs, es,
                eW1a, eW1b, eW1c, eb1, eW2, eb2, nW1a, nW1b, nb1, nW2, nb2):
    rcon = _bdot(pn, eW1b)
    nes = []
    for sf, e in zip(sfs, es):
        xk = _bdot(sf, eW1a) + rcon + _bdot(e, eW1c) + eb1
        nes.append(_bdot(_gelu(xk), eW2) + eb2)
    agg = (nes[0] + nes[1]) + nes[2]
    nx = _bdot(pn, nW1a) + _bdot(agg, nW1b) + nb1
    nn = _bdot(_gelu(nx), nW2) + nb2
    return pn + nn, nes


def _layer_body(pn_ref, s0_ref, s1_ref, s2_ref, e0_ref, e1_ref, e2_ref,
                eW1a_ref, eW1b_ref, eW1c_ref, eb1_ref, eW2_ref, eb2_ref,
                nW1a_ref, nW1b_ref, nb1_ref, nW2_ref, nb2_ref,
                pno_ref, eo0_ref, eo1_ref, eo2_ref):
    es = (e0_ref[...], e1_ref[...], e2_ref[...])
    pno, nes = _layer_core(
        pn_ref[...], (s0_ref[...], s1_ref[...], s2_ref[...]), es,
        eW1a_ref[...], eW1b_ref[...], eW1c_ref[...], eb1_ref[...],
        eW2_ref[...], eb2_ref[...], nW1a_ref[...], nW1b_ref[...],
        nb1_ref[...], nW2_ref[...], nb2_ref[...])
    pno_ref[...] = pno
    for e, ne, eo_ref in zip(es, nes, (eo0_ref, eo1_ref, eo2_ref)):
        eo_ref[...] = e + ne


def _layer_out_body(pn_ref, s0_ref, s1_ref, s2_ref, e0_ref, e1_ref, e2_ref,
                    eW1a_ref, eW1b_ref, eW1c_ref, eb1_ref, eW2_ref, eb2_ref,
                    nW1a_ref, nW1b_ref, nb1_ref, nW2_ref, nb2_ref,
                    oW1_ref, ob1_ref, oW2_ref, ob2_ref, oW3_ref, ob3_ref,
                    o_ref):
    pno, _ = _layer_core(
        pn_ref[...], (s0_ref[...], s1_ref[...], s2_ref[...]),
        (e0_ref[...], e1_ref[...], e2_ref[...]),
        eW1a_ref[...], eW1b_ref[...], eW1c_ref[...], eb1_ref[...],
        eW2_ref[...], eb2_ref[...], nW1a_ref[...], nW1b_ref[...],
        nb1_ref[...], nW2_ref[...], nb2_ref[...])
    h = _gelu(_bdot(pno, oW1_ref[...]) + ob1_ref[...])
    h = _gelu(_bdot(h, oW2_ref[...]) + ob2_ref[...])
    o_ref[...] = _bdot(h, oW3_ref[...]) + ob3_ref[...]


def _full(shape):
    nd = len(shape)
    return pl.BlockSpec(shape, lambda i: (0,) * nd)


def _row(cols):
    return pl.BlockSpec((BD, cols), lambda i: (i, 0))


_GRID = (NQP // BD,)


def _pre(gs, ndb, wts):
    return pl.pallas_call(
        _pre_body,
        grid=_GRID,
        in_specs=[_row(ND)] * 3 + [pl.BlockSpec((BD, KNN), lambda i: (i, 0))]
                 + [_full(w.shape) for w in wts],
        out_specs=[_row(ND), _row(ED), _row(ED), _row(ED)],
        out_shape=[jax.ShapeDtypeStruct((NQP, ND), jnp.float32)]
                  + [jax.ShapeDtypeStruct((NQP, ED), jnp.float32)] * 3,
    )(*gs, ndb, *wts)


def _layer(pn, sfs, es, wts):
    return pl.pallas_call(
        _layer_body,
        grid=_GRID,
        in_specs=[_row(ND)] * 4 + [_row(ED)] * 3
                 + [_full(w.shape) for w in wts],
        out_specs=[_row(ND), _row(ED), _row(ED), _row(ED)],
        out_shape=[jax.ShapeDtypeStruct((NQP, ND), jnp.float32)]
                  + [jax.ShapeDtypeStruct((NQP, ED), jnp.float32)] * 3,
    )(pn, *sfs, *es, *wts)


def _layer_out(pn, sfs, es, wts, owts):
    return pl.pallas_call(
        _layer_out_body,
        grid=_GRID,
        in_specs=[_row(ND)] * 4 + [_row(ED)] * 3
                 + [_full(w.shape) for w in wts]
                 + [_full(w.shape) for w in owts],
        out_specs=pl.BlockSpec((BD, 3), lambda i: (i, 0)),
        out_shape=jax.ShapeDtypeStruct((NQP, 3), jnp.float32),
    )(pn, *sfs, *es, *wts, *owts)


def _layer_wts(lp, bf):
    eW1, eW2 = lp["eW"]
    eb1, eb2 = lp["eb"]
    nW1, nW2 = lp["nW"]
    nb1, nb2 = lp["nb"]
    return (eW1[:ND].astype(bf), eW1[ND:2 * ND].astype(bf),
            eW1[2 * ND:].astype(bf), eb1.reshape(1, ND),
            eW2.astype(bf), eb2.reshape(1, ED),
            nW1[:ND].astype(bf), nW1[ND:].astype(bf), nb1.reshape(1, ND),
            nW2.astype(bf), nb2.reshape(1, ND))


def kernel(nodes, positions, query_positions, params):
    bf = jnp.bfloat16
    f32 = jnp.float32
    idx, nd = _knn(query_positions, positions)
    senders = tuple(idx[:, k] for k in range(KNN))

    encW1, encW2 = params["enc_W"]
    encb1, encb2 = params["enc_b"]
    w1s = jnp.sum(encW1.astype(bf).astype(f32), axis=0).reshape(1, ND)
    ndb = nd.astype(bf)
    pre_wts = (w1s, encb1.reshape(1, ND), encW2.astype(bf),
               encb2.reshape(1, ED))

    gs = _sc_gather(nodes, *senders)
    pn, e0, e1, e2 = _pre(gs, ndb, pre_wts)

    lps = params["layers"]
    sfs = _sc_gather(pn, *senders)
    pn, e0, e1, e2 = _layer(pn, sfs, (e0, e1, e2), _layer_wts(lps[0], bf))

    oW1, oW2, oW3 = params["out_W"]
    ob1, ob2, ob3 = params["out_b"]
    owts = (oW1.astype(bf), ob1.reshape(1, ND),
            oW2.astype(bf), ob2.reshape(1, ND),
            oW3.astype(bf), ob3.reshape(1, 3))
    sfs = _sc_gather(pn, *senders)
    out = _layer_out(pn, sfs, (e0, e1, e2), _layer_wts(lps[1], bf), owts)
    return out[:NQ]


# speedup vs baseline: 8.4274x; 1.0199x over previous
import functools

import jax
import jax.numpy as jnp
from jax import lax
from jax.experimental import pallas as pl
from jax.experimental.pallas import tpu as pltpu
from jax.experimental.pallas import tpu_sc as plsc

KNN = 3
NQ = 10000
NS = 10000
NQP = 10240
NSP = 10240
BQ = 512
BD = 512
ND = 128
ED = 16


def _knn_body(qb_ref, pt2_ref, q2_ref, p2_ref, idx_ref, nd_ref):
    qb = qb_ref[...]
    pt2 = pt2_ref[...]
    qp2 = jax.lax.dot_general(qb, pt2, (((1,), (0,)), ((), ())),
                              preferred_element_type=jnp.float32)
    d2 = (q2_ref[...] + p2_ref[...]) - qp2
    x = jnp.maximum(d2, 1e-12)
    col = jax.lax.broadcasted_iota(jnp.int32, (BQ, NSP), 1)
    idxs, nds = [], []
    for k in range(KNN):
        m = jnp.min(x, axis=1, keepdims=True)
        j = jnp.min(jnp.where(x == m, col, NSP), axis=1, keepdims=True)
        idxs.append(j)
        nds.append(jnp.sqrt(m))
        if k + 1 < KNN:
            x = jnp.where(col == j, jnp.float32(jnp.inf), x)
    idx_ref[...] = jnp.concatenate(idxs, axis=1)
    nd_ref[...] = jnp.concatenate(nds, axis=1)


def _knn(query_positions, positions):
    qb = jnp.pad(query_positions.astype(jnp.bfloat16),
                 ((0, NQP - NQ), (0, 5)))
    pt2 = jnp.pad(positions.astype(jnp.bfloat16) * 2,
                  ((0, NSP - NS), (0, 5))).T
    q2 = jnp.pad(jnp.sum(query_positions ** 2, axis=1),
                 (0, NQP - NQ)).reshape(NQP, 1)
    p2 = jnp.pad(jnp.sum(positions ** 2, axis=1),
                 (0, NSP - NS), constant_values=1e12).reshape(1, NSP)
    return pl.pallas_call(
        _knn_body,
        grid=(NQP // BQ,),
        in_specs=[
            pl.BlockSpec((BQ, 8), lambda i: (i, 0)),
            pl.BlockSpec((8, NSP), lambda i: (0, 0)),
            pl.BlockSpec((BQ, 1), lambda i: (i, 0)),
            pl.BlockSpec((1, NSP), lambda i: (0, 0)),
        ],
        out_specs=[
            pl.BlockSpec((BQ, KNN), lambda i: (i, 0)),
            pl.BlockSpec((BQ, KNN), lambda i: (i, 0)),
        ],
        out_shape=[
            jax.ShapeDtypeStruct((NQP, KNN), jnp.int32),
            jax.ShapeDtypeStruct((NQP, KNN), jnp.float32),
        ],
    )(qb, pt2, q2, p2)


_NW = 32
_BPW = NQP // _NW
_CHUNK = 80


@functools.partial(
    pl.kernel,
    mesh=plsc.VectorSubcoreMesh(core_axis_name="c", subcore_axis_name="s"),
    out_type=[jax.ShapeDtypeStruct((NQP, ND), jnp.float32)] * KNN,
    scratch_types=[
        pltpu.VMEM((KNN * _BPW,), jnp.int32),
        pltpu.VMEM((KNN * _BPW, ND), jnp.float32),
        pltpu.SemaphoreType.DMA,
        pltpu.SemaphoreType.DMA,
        pltpu.SemaphoreType.DMA,
        pltpu.SemaphoreType.DMA,
    ],
)
def _sc_gather(table_hbm, i0_hbm, i1_hbm, i2_hbm,
               o0_hbm, o1_hbm, o2_hbm, idx_v, rows_v, sem_i, s0, s1, s2):
    wid = lax.axis_index("s") * 2 + lax.axis_index("c")
    base = wid * _BPW
    sems = (s0, s1, s2)
    icopies = [
        pltpu.async_copy(idx_hbm.at[pl.ds(base, _BPW)],
                         idx_v.at[pl.ds(k * _BPW, _BPW)], sem_i)
        for k, idx_hbm in enumerate((i0_hbm, i1_hbm, i2_hbm))
    ]
    for c in icopies:
        c.wait()
    copies = [
        [pltpu.async_copy(
            table_hbm.at[idx_v.at[pl.ds(k * _BPW + j * _CHUNK, _CHUNK)]],
            rows_v.at[pl.ds(k * _BPW + j * _CHUNK, _CHUNK)], sems[k])
         for j in range(_BPW // _CHUNK)]
        for k in range(KNN)
    ]
    for k, out_hbm in enumerate((o0_hbm, o1_hbm, o2_hbm)):
        for c in copies[k]:
            c.wait()
        pltpu.sync_copy(rows_v.at[pl.ds(k * _BPW, _BPW)],
                        out_hbm.at[pl.ds(base, _BPW)])


_SQRT_HALF = 0.7071067811865476


def _gelu(x):
    return 0.5 * x * (1.0 + jax.lax.erf(x * _SQRT_HALF))


def _bdot(a, b):
    return jax.lax.dot_general(a.astype(jnp.bfloat16), b,
                               (((1,), (0,)), ((), ())),
                               preferred_element_type=jnp.float32)


def _pre_body(g0_ref, g1_ref, g2_ref, ndb_ref, w1s_ref, b1_ref, w2_ref,
              b2_ref, pn_ref, e0_ref, e1_ref, e2_ref):
    s = (g0_ref[...] + g1_ref[...]) + g2_ref[...]
    pn_ref[...] = s / 3.0
    ndb = ndb_ref[...].astype(jnp.float32)
    w1s = w1s_ref[...]
    for k, e_ref in enumerate((e0_ref, e1_ref, e2_ref)):
        h = _gelu(ndb[:, k:k + 1] * w1s + b1_ref[...])
        e_ref[...] = _bdot(h, w2_ref[...]) + b2_ref[...]


def _layer_core(pn, sfs, es,
                eW1a, eW1b, eW1c, eb1, eW2, eb2, nW1a, nW1b, nb1, nW2, nb2):
    rcon = _bdot(pn, eW1b)
    nes = []
    for sf, e in zip(sfs, es):
        xk = _bdot(sf, eW1a) + rcon + _bdot(e, eW1c) + eb1
        nes.append(_bdot(_gelu(xk), eW2) + eb2)
    agg = (nes[0] + nes[1]) + nes[2]
    nx = _bdot(pn, nW1a) + _bdot(agg, nW1b) + nb1
    nn = _bdot(_gelu(nx), nW2) + nb2
    return pn + nn, nes


def _layer_body(pn_ref, s0_ref, s1_ref, s2_ref, e0_ref, e1_ref, e2_ref,
                eW1a_ref, eW1b_ref, eW1c_ref, eb1_ref, eW2_ref, eb2_ref,
                nW1a_ref, nW1b_ref, nb1_ref, nW2_ref, nb2_ref,
                pno_ref, eo0_ref, eo1_ref, eo2_ref):
    es = (e0_ref[...], e1_ref[...], e2_ref[...])
    pno, nes = _layer_core(
        pn_ref[...], (s0_ref[...], s1_ref[...], s2_ref[...]), es,
        eW1a_ref[...], eW1b_ref[...], eW1c_ref[...], eb1_ref[...],
        eW2_ref[...], eb2_ref[...], nW1a_ref[...], nW1b_ref[...],
        nb1_ref[...], nW2_ref[...], nb2_ref[...])
    pno_ref[...] = pno
    for e, ne, eo_ref in zip(es, nes, (eo0_ref, eo1_ref, eo2_ref)):
        eo_ref[...] = e + ne


def _layer_out_body(pn_ref, s0_ref, s1_ref, s2_ref, e0_ref, e1_ref, e2_ref,
                    eW1a_ref, eW1b_ref, eW1c_ref, eb1_ref, eW2_ref, eb2_ref,
                    nW1a_ref, nW1b_ref, nb1_ref, nW2_ref, nb2_ref,
                    oW1_ref, ob1_ref, oW2_ref, ob2_ref, oW3_ref, ob3_ref,
                    o_ref):
    pno, _ = _layer_core(
        pn_ref[...], (s0_ref[...], s1_ref[...], s2_ref[...]),
        (e0_ref[...], e1_ref[...], e2_ref[...]),
        eW1a_ref[...], eW1b_ref[...], eW1c_ref[...], eb1_ref[...],
        eW2_ref[...], eb2_ref[...], nW1a_ref[...], nW1b_ref[...],
        nb1_ref[...], nW2_ref[...], nb2_ref[...])
    h = _gelu(_bdot(pno, oW1_ref[...]) + ob1_ref[...])
    h = _gelu(_bdot(h, oW2_ref[...]) + ob2_ref[...])
    o_ref[...] = _bdot(h, oW3_ref[...]) + ob3_ref[...]


def _full(shape):
    nd = len(shape)
    return pl.BlockSpec(shape, lambda i: (0,) * nd)


def _row(cols):
    return pl.BlockSpec((BD, cols), lambda i: (i, 0))


_GRID = (NQP // BD,)


def _pre(gs, ndb, wts):
    return pl.pallas_call(
        _pre_body,
        grid=_GRID,
        in_specs=[_row(ND)] * 3 + [pl.BlockSpec((BD, KNN), lambda i: (i, 0))]
                 + [_full(w.shape) for w in wts],
        out_specs=[_row(ND), _row(ED), _row(ED), _row(ED)],
        out_shape=[jax.ShapeDtypeStruct((NQP, ND), jnp.float32)]
                  + [jax.ShapeDtypeStruct((NQP, ED), jnp.float32)] * 3,
    )(*gs, ndb, *wts)


def _layer(pn, sfs, es, wts):
    return pl.pallas_call(
        _layer_body,
        grid=_GRID,
        in_specs=[_row(ND)] * 4 + [_row(ED)] * 3
                 + [_full(w.shape) for w in wts],
        out_specs=[_row(ND), _row(ED), _row(ED), _row(ED)],
        out_shape=[jax.ShapeDtypeStruct((NQP, ND), jnp.float32)]
                  + [jax.ShapeDtypeStruct((NQP, ED), jnp.float32)] * 3,
    )(pn, *sfs, *es, *wts)


def _layer_out(pn, sfs, es, wts, owts):
    return pl.pallas_call(
        _layer_out_body,
        grid=_GRID,
        in_specs=[_row(ND)] * 4 + [_row(ED)] * 3
                 + [_full(w.shape) for w in wts]
                 + [_full(w.shape) for w in owts],
        out_specs=pl.BlockSpec((BD, 3), lambda i: (i, 0)),
        out_shape=jax.ShapeDtypeStruct((NQP, 3), jnp.float32),
    )(pn, *sfs, *es, *wts, *owts)


def _layer_wts(lp, bf):
    eW1, eW2 = lp["eW"]
    eb1, eb2 = lp["eb"]
    nW1, nW2 = lp["nW"]
    nb1, nb2 = lp["nb"]
    return (eW1[:ND].astype(bf), eW1[ND:2 * ND].astype(bf),
            eW1[2 * ND:].astype(bf), eb1.reshape(1, ND),
            eW2.astype(bf), eb2.reshape(1, ED),
            nW1[:ND].astype(bf), nW1[ND:].astype(bf), nb1.reshape(1, ND),
            nW2.astype(bf), nb2.reshape(1, ND))


def kernel(nodes, positions, query_positions, params):
    bf = jnp.bfloat16
    f32 = jnp.float32
    idx, nd = _knn(query_positions, positions)
    senders = tuple(idx[:, k] for k in range(KNN))

    encW1, encW2 = params["enc_W"]
    encb1, encb2 = params["enc_b"]
    w1s = jnp.sum(encW1.astype(bf).astype(f32), axis=0).reshape(1, ND)
    ndb = nd.astype(bf)
    pre_wts = (w1s, encb1.reshape(1, ND), encW2.astype(bf),
               encb2.reshape(1, ED))

    gs = _sc_gather(nodes, *senders)
    pn, e0, e1, e2 = _pre(gs, ndb, pre_wts)

    lps = params["layers"]
    sfs = _sc_gather(pn, *senders)
    pn, e0, e1, e2 = _layer(pn, sfs, (e0, e1, e2), _layer_wts(lps[0], bf))

    oW1, oW2, oW3 = params["out_W"]
    ob1, ob2, ob3 = params["out_b"]
    owts = (oW1.astype(bf), ob1.reshape(1, ND),
            oW2.astype(bf), ob2.reshape(1, ND),
            oW3.astype(bf), ob3.reshape(1, 3))
    sfs = _sc_gather(pn, *senders)
    out = _layer_out(pn, sfs, (e0, e1, e2), _layer_wts(lps[1], bf), owts)
    return out[:NQ]
